# Initial kernel scaffold; baseline (speedup 1.0000x reference)
import functools

import jax
import jax.numpy as jnp
from jax import lax
from jax.experimental import pallas as pl
from jax.experimental.pallas import tpu as pltpu
from jax.experimental.pallas import tpu_sc as plsc

N = 10000
NPAD = 10240
D = 128
E = 320000
NC = 2
NS = 16
NW = NC * NS
CHUNK = 128
NCHUNKS = 79
EPT = CHUNK * NCHUNKS
EPAD = NW * EPT
SLAB = NPAD // NS
DEGW = 16
BLK = 1024


def _sc_mesh():
    return plsc.VectorSubcoreMesh(core_axis_name="c", subcore_axis_name="s")


def _deg_kernel(dst3):
    @functools.partial(
        pl.kernel,
        out_type=jax.ShapeDtypeStruct((NC, NPAD, DEGW), jnp.float32),
        mesh=_sc_mesh(),
        scratch_types=[
            pltpu.VMEM((CHUNK,), jnp.int32),
            pltpu.VMEM((CHUNK, DEGW), jnp.float32),
            pltpu.VMEM((CHUNK, DEGW), jnp.float32),
            pltpu.VMEM_SHARED((NPAD, DEGW), jnp.float32),
        ],
    )
    def k(dst_ref, out_ref, idx_v, ones_v, zeros_v, acc):
        c = lax.axis_index("c")
        s = lax.axis_index("s")
        wid = c * NS + s

        @pl.loop(0, CHUNK)
        def _(i):
            ones_v[i, :] = jnp.full((DEGW,), 1.0, jnp.float32)
            zeros_v[i, :] = jnp.zeros((DEGW,), jnp.float32)

        @pl.loop(0, SLAB // CHUNK)
        def _(j):
            pltpu.sync_copy(zeros_v, acc.at[pl.ds(s * SLAB + j * CHUNK, CHUNK)])

        plsc.subcore_barrier()

        @pl.loop(0, NCHUNKS)
        def _(ch):
            pltpu.sync_copy(dst_ref.at[wid, ch], idx_v)
            pltpu.sync_copy(ones_v, acc.at[idx_v], add=True)

        plsc.subcore_barrier()
        pltpu.sync_copy(acc.at[pl.ds(s * SLAB, SLAB)],
                        out_ref.at[c, pl.ds(s * SLAB, SLAB)])

    return k(dst3)


def _rowpass_kernel(hs, src3, dst3):
    @functools.partial(
        pl.kernel,
        out_type=jax.ShapeDtypeStruct((NC, NPAD, D), jnp.float32),
        mesh=_sc_mesh(),
        scratch_types=[
            pltpu.VMEM((CHUNK,), jnp.int32),
            pltpu.VMEM((CHUNK,), jnp.int32),
            pltpu.VMEM((CHUNK, D), jnp.float32),
            pltpu.VMEM_SHARED((NPAD, D), jnp.float32),
            pltpu.SemaphoreType.DMA,
        ],
    )
    def k(hs_ref, src_ref, dst_ref, out_ref, src_v, dst_v, rows_v, acc, sem):
        c = lax.axis_index("c")
        s = lax.axis_index("s")
        wid = c * NS + s

        @pl.loop(0, CHUNK)
        def _(i):
            @pl.loop(0, D // 16)
            def _(j):
                rows_v[i, pl.ds(j * 16, 16)] = jnp.zeros((16,), jnp.float32)

        @pl.loop(0, SLAB // CHUNK)
        def _(j):
            pltpu.sync_copy(rows_v, acc.at[pl.ds(s * SLAB + j * CHUNK, CHUNK)])

        plsc.subcore_barrier()

        @pl.loop(0, NCHUNKS)
        def _(ch):
            pltpu.sync_copy(src_ref.at[wid, ch], src_v)
            pltpu.sync_copy(dst_ref.at[wid, ch], dst_v)
            pltpu.async_copy(hs_ref.at[src_v], rows_v, sem).wait()
            pltpu.sync_copy(rows_v, acc.at[dst_v], add=True)

        plsc.subcore_barrier()
        pltpu.sync_copy(acc.at[pl.ds(s * SLAB, SLAB)],
                        out_ref.at[c, pl.ds(s * SLAB, SLAB)])

    return k(hs, src3, dst3)


def _matmul_kernel(x, w):
    def body(x_ref, w_ref, o_ref):
        o_ref[...] = jnp.dot(x_ref[...], w_ref[...],
                             preferred_element_type=jnp.float32)

    return pl.pallas_call(
        body,
        grid=(NPAD // BLK,),
        in_specs=[
            pl.BlockSpec((BLK, D), lambda i: (i, 0)),
            pl.BlockSpec((D, D), lambda i: (0, 0)),
        ],
        out_specs=pl.BlockSpec((BLK, D), lambda i: (i, 0)),
        out_shape=jax.ShapeDtypeStruct((NPAD, D), jnp.float32),
    )(x, w)


def _scale_kernel(deg, h):
    def body(deg_ref, h_ref, dinv_ref, hs_ref):
        degsum = deg_ref[0, :, 0:1] + deg_ref[1, :, 0:1] + 1.0
        dinv = lax.rsqrt(degsum)
        dinv_rep = jnp.broadcast_to(dinv, (BLK, D))
        dinv_ref[...] = dinv_rep
        hs_ref[...] = dinv_rep * h_ref[...]

    return pl.pallas_call(
        body,
        grid=(NPAD // BLK,),
        in_specs=[
            pl.BlockSpec((NC, BLK, DEGW), lambda i: (0, i, 0)),
            pl.BlockSpec((BLK, D), lambda i: (i, 0)),
        ],
        out_specs=[
            pl.BlockSpec((BLK, D), lambda i: (i, 0)),
            pl.BlockSpec((BLK, D), lambda i: (i, 0)),
        ],
        out_shape=[
            jax.ShapeDtypeStruct((NPAD, D), jnp.float32),
            jax.ShapeDtypeStruct((NPAD, D), jnp.float32),
        ],
    )(deg, h)


def _layer_out_kernel(sp, hs, dinv, b, w, scale_out):
    def body(sp_ref, hs_ref, dinv_ref, b_ref, w_ref, o_ref):
        t = dinv_ref[...] * (sp_ref[0] + sp_ref[1] + hs_ref[...]) + b_ref[...]
        r = jnp.maximum(t, 0.0)
        o = jnp.dot(r, w_ref[...], preferred_element_type=jnp.float32)
        if scale_out:
            o = dinv_ref[...] * o
        o_ref[...] = o

    return pl.pallas_call(
        body,
        grid=(NPAD // BLK,),
        in_specs=[
            pl.BlockSpec((NC, BLK, D), lambda i: (0, i, 0)),
            pl.BlockSpec((BLK, D), lambda i: (i, 0)),
            pl.BlockSpec((BLK, D), lambda i: (i, 0)),
            pl.BlockSpec((1, D), lambda i: (0, 0)),
            pl.BlockSpec((D, D), lambda i: (0, 0)),
        ],
        out_specs=pl.BlockSpec((BLK, D), lambda i: (i, 0)),
        out_shape=jax.ShapeDtypeStruct((NPAD, D), jnp.float32),
    )(sp, hs, dinv, b, w)


def kernel(x, edge_index, W1, b1, W2, b2, Wt, bt, We, be):
    src = edge_index[0].astype(jnp.int32)
    dst = edge_index[1].astype(jnp.int32)
    pad = (jnp.arange(EPAD - E, dtype=jnp.int32) % (NPAD - N)) + N
    src3 = jnp.concatenate([src, pad]).reshape(NW, NCHUNKS, CHUNK)
    dst3 = jnp.concatenate([dst, pad]).reshape(NW, NCHUNKS, CHUNK)

    xp = jnp.zeros((NPAD, D), jnp.float32).at[:N].set(x)
    b1r = b1.reshape(1, D)
    Wh = jnp.zeros((D, D), jnp.float32).at[:, 0:1].set(Wt).at[:, 1:2].set(We)
    bh = jnp.zeros((1, D), jnp.float32).at[0, 0].set(bt[0]).at[0, 1].set(be[0])

    deg = _deg_kernel(dst3)
    h1 = _matmul_kernel(xp, W1)
    dinv, hs1 = _scale_kernel(deg, h1)
    s1 = _rowpass_kernel(hs1, src3, dst3)
    hs2 = _layer_out_kernel(s1, hs1, dinv, b1r, W2, True)
    s2 = _rowpass_kernel(hs2, src3, dst3)
    out = _layer_out_kernel(s2, hs2, dinv, bh, Wh, False)

    return (out[:N, 0:1], out[:N, 1:2])


# speedup vs baseline: 15.8610x; 15.8610x over previous
import functools

import jax
import jax.numpy as jnp
from jax import lax
from jax.experimental import pallas as pl
from jax.experimental.pallas import tpu as pltpu
from jax.experimental.pallas import tpu_sc as plsc

N = 10000
NPAD = 10240
D = 128
E = 320000
NC = 2
NS = 16
NW = NC * NS
CHUNK = 128
NCHUNKS = 79
EPT = CHUNK * NCHUNKS
EPAD = NW * EPT
SLAB = NPAD // NS
DEGW = 16
BLK = 1024


def _sc_mesh():
    return plsc.VectorSubcoreMesh(core_axis_name="c", subcore_axis_name="s")


def _deg_kernel(dst3):
    @functools.partial(
        pl.kernel,
        out_type=jax.ShapeDtypeStruct((NC, NPAD, DEGW), jnp.float32),
        mesh=_sc_mesh(),
        scratch_types=[
            pltpu.VMEM((CHUNK,), jnp.int32),
            pltpu.VMEM((CHUNK, DEGW), jnp.float32),
            pltpu.VMEM((CHUNK, DEGW), jnp.float32),
            pltpu.VMEM_SHARED((NPAD, DEGW), jnp.float32),
        ],
    )
    def k(dst_ref, out_ref, idx_v, ones_v, zeros_v, acc):
        c = lax.axis_index("c")
        s = lax.axis_index("s")
        wid = c * NS + s

        @pl.loop(0, CHUNK)
        def _(i):
            ones_v[i, :] = jnp.full((DEGW,), 1.0, jnp.float32)
            zeros_v[i, :] = jnp.zeros((DEGW,), jnp.float32)

        @pl.loop(0, SLAB // CHUNK)
        def _(j):
            pltpu.sync_copy(zeros_v, acc.at[pl.ds(s * SLAB + j * CHUNK, CHUNK)])

        plsc.subcore_barrier()

        @pl.loop(0, NCHUNKS)
        def _(ch):
            pltpu.sync_copy(dst_ref.at[wid, ch], idx_v)
            pltpu.sync_copy(ones_v, acc.at[idx_v], add=True)

        plsc.subcore_barrier()
        pltpu.sync_copy(acc.at[pl.ds(s * SLAB, SLAB)],
                        out_ref.at[c, pl.ds(s * SLAB, SLAB)])

    return k(dst3)


def _rowpass_kernel(hs, src3, dst3):
    @functools.partial(
        pl.kernel,
        out_type=jax.ShapeDtypeStruct((NC, NPAD, D), jnp.float32),
        mesh=_sc_mesh(),
        scratch_types=[
            pltpu.VMEM((CHUNK,), jnp.int32),
            pltpu.VMEM((CHUNK,), jnp.int32),
            pltpu.VMEM((CHUNK, D), jnp.float32),
            pltpu.VMEM_SHARED((NPAD, D), jnp.float32),
            pltpu.SemaphoreType.DMA,
        ],
    )
    def k(hs_ref, src_ref, dst_ref, out_ref, src_v, dst_v, rows_v, acc, sem):
        c = lax.axis_index("c")
        s = lax.axis_index("s")
        wid = c * NS + s

        @pl.loop(0, CHUNK)
        def _(i):
            @pl.loop(0, D // 16)
            def _(j):
                rows_v[i, pl.ds(j * 16, 16)] = jnp.zeros((16,), jnp.float32)

        @pl.loop(0, SLAB // CHUNK)
        def _(j):
            pltpu.sync_copy(rows_v, acc.at[pl.ds(s * SLAB + j * CHUNK, CHUNK)])

        plsc.subcore_barrier()

        @pl.loop(0, NCHUNKS)
        def _(ch):
            pltpu.sync_copy(src_ref.at[wid, ch], src_v)
            pltpu.sync_copy(dst_ref.at[wid, ch], dst_v)
            pltpu.async_copy(hs_ref.at[src_v], rows_v, sem).wait()
            pltpu.sync_copy(rows_v, acc.at[dst_v], add=True)

        plsc.subcore_barrier()
        pltpu.sync_copy(acc.at[pl.ds(s * SLAB, SLAB)],
                        out_ref.at[c, pl.ds(s * SLAB, SLAB)])

    return k(hs, src3, dst3)


def _matmul_kernel(x, w):
    def body(x_ref, w_ref, o_ref):
        o_ref[...] = jnp.dot(x_ref[...], w_ref[...],
                             preferred_element_type=jnp.float32)

    return pl.pallas_call(
        body,
        grid=(NPAD // BLK,),
        in_specs=[
            pl.BlockSpec((BLK, D), lambda i: (i, 0)),
            pl.BlockSpec((D, D), lambda i: (0, 0)),
        ],
        out_specs=pl.BlockSpec((BLK, D), lambda i: (i, 0)),
        out_shape=jax.ShapeDtypeStruct((NPAD, D), jnp.float32),
    )(x, w)


def _scale_kernel(deg, h):
    def body(deg_ref, h_ref, dinv_ref, hs_ref):
        degsum = deg_ref[0, :, 0:1] + deg_ref[1, :, 0:1] + 1.0
        dinv = lax.rsqrt(degsum)
        dinv_rep = jnp.broadcast_to(dinv, (BLK, D))
        dinv_ref[...] = dinv_rep
        hs_ref[...] = dinv_rep * h_ref[...]

    return pl.pallas_call(
        body,
        grid=(NPAD // BLK,),
        in_specs=[
            pl.BlockSpec((NC, BLK, DEGW), lambda i: (0, i, 0)),
            pl.BlockSpec((BLK, D), lambda i: (i, 0)),
        ],
        out_specs=[
            pl.BlockSpec((BLK, D), lambda i: (i, 0)),
            pl.BlockSpec((BLK, D), lambda i: (i, 0)),
        ],
        out_shape=[
            jax.ShapeDtypeStruct((NPAD, D), jnp.float32),
            jax.ShapeDtypeStruct((NPAD, D), jnp.float32),
        ],
    )(deg, h)


def _layer_out_kernel(sp, hs, dinv, pre_b, w, post_b, scale_out):
    def body(sp_ref, hs_ref, dinv_ref, pb_ref, w_ref, qb_ref, o_ref):
        t = dinv_ref[...] * (sp_ref[0] + sp_ref[1] + hs_ref[...]) + pb_ref[...]
        r = jnp.maximum(t, 0.0)
        o = jnp.dot(r, w_ref[...], preferred_element_type=jnp.float32)
        if scale_out:
            o = dinv_ref[...] * o
        o_ref[...] = o + qb_ref[...]

    return pl.pallas_call(
        body,
        grid=(NPAD // BLK,),
        in_specs=[
            pl.BlockSpec((NC, BLK, D), lambda i: (0, i, 0)),
            pl.BlockSpec((BLK, D), lambda i: (i, 0)),
            pl.BlockSpec((BLK, D), lambda i: (i, 0)),
            pl.BlockSpec((1, D), lambda i: (0, 0)),
            pl.BlockSpec((D, D), lambda i: (0, 0)),
            pl.BlockSpec((1, D), lambda i: (0, 0)),
        ],
        out_specs=pl.BlockSpec((BLK, D), lambda i: (i, 0)),
        out_shape=jax.ShapeDtypeStruct((NPAD, D), jnp.float32),
    )(sp, hs, dinv, pre_b, w, post_b)


def kernel(x, edge_index, W1, b1, W2, b2, Wt, bt, We, be):
    src = edge_index[0].astype(jnp.int32)
    dst = edge_index[1].astype(jnp.int32)
    pad = (jnp.arange(EPAD - E, dtype=jnp.int32) % (NPAD - N)) + N
    src3 = jnp.concatenate([src, pad]).reshape(NW, NCHUNKS, CHUNK)
    dst3 = jnp.concatenate([dst, pad]).reshape(NW, NCHUNKS, CHUNK)

    xp = jnp.zeros((NPAD, D), jnp.float32).at[:N].set(x)
    b1r = b1.reshape(1, D)
    b2r = b2.reshape(1, D)
    zb = jnp.zeros((1, D), jnp.float32)
    Wh = jnp.zeros((D, D), jnp.float32).at[:, 0:1].set(Wt).at[:, 1:2].set(We)
    bh = jnp.zeros((1, D), jnp.float32).at[0, 0].set(bt[0]).at[0, 1].set(be[0])

    deg = _deg_kernel(dst3)
    h1 = _matmul_kernel(xp, W1)
    dinv, hs1 = _scale_kernel(deg, h1)
    s1 = _rowpass_kernel(hs1, src3, dst3)
    hs2 = _layer_out_kernel(s1, hs1, dinv, b1r, W2, zb, True)
    s2 = _rowpass_kernel(hs2, src3, dst3)
    out = _layer_out_kernel(s2, hs2, dinv, b2r, Wh, bh, False)

    return (out[:N, 0:1], out[:N, 1:2])


# speedup vs baseline: 23.6720x; 1.4925x over previous
import functools

import jax
import jax.numpy as jnp
from jax import lax
from jax.experimental import pallas as pl
from jax.experimental.pallas import tpu as pltpu
from jax.experimental.pallas import tpu_sc as plsc

N = 10000
NPAD = 10240
D = 128
E = 320000
NC = 2
NS = 16
NW = NC * NS
CHUNK = 128
NCHUNKS = 80
NBUF = 4
EPT = CHUNK * NCHUNKS
EPAD = NW * EPT
SLAB = NPAD // NS
DEGW = 16
BLK = 1024


def _sc_mesh():
    return plsc.VectorSubcoreMesh(core_axis_name="c", subcore_axis_name="s")


def _deg_kernel(dst3):
    @functools.partial(
        pl.kernel,
        out_type=jax.ShapeDtypeStruct((NC, NPAD, DEGW), jnp.float32),
        mesh=_sc_mesh(),
        scratch_types=[
            pltpu.VMEM((CHUNK,), jnp.int32),
            pltpu.VMEM((CHUNK,), jnp.int32),
            pltpu.VMEM((CHUNK, DEGW), jnp.float32),
            pltpu.VMEM((CHUNK, DEGW), jnp.float32),
            pltpu.VMEM_SHARED((NPAD, DEGW), jnp.float32),
            pltpu.SemaphoreType.DMA,
            pltpu.SemaphoreType.DMA,
        ],
    )
    def k(dst_ref, out_ref, iv0, iv1, ones_v, zeros_v, acc, is0, is1):
        idx_v = (iv0, iv1)
        isems = (is0, is1)
        c = lax.axis_index("c")
        s = lax.axis_index("s")
        wid = c * NS + s

        def fire_idx(g, sl):
            pltpu.async_copy(dst_ref.at[wid, g], idx_v[sl], isems[sl])

        def wait_idx(g, sl):
            pltpu.make_async_copy(dst_ref.at[wid, g], idx_v[sl],
                                  isems[sl]).wait()

        fire_idx(0, 0)
        fire_idx(1, 1)

        @pl.loop(0, CHUNK)
        def _(i):
            ones_v[i, :] = jnp.full((DEGW,), 1.0, jnp.float32)
            zeros_v[i, :] = jnp.zeros((DEGW,), jnp.float32)

        @pl.loop(0, SLAB // CHUNK)
        def _(j):
            pltpu.sync_copy(zeros_v, acc.at[pl.ds(s * SLAB + j * CHUNK, CHUNK)])

        plsc.subcore_barrier()

        @pl.loop(0, NCHUNKS, step=2)
        def _(c0):
            for b in range(2):
                g = c0 + b
                wait_idx(g, b)
                pltpu.sync_copy(ones_v, acc.at[idx_v[b]], add=True)

                @pl.when(g + 2 < NCHUNKS)
                def _():
                    fire_idx(g + 2, b)

        plsc.subcore_barrier()
        pltpu.sync_copy(acc.at[pl.ds(s * SLAB, SLAB)],
                        out_ref.at[c, pl.ds(s * SLAB, SLAB)])

    return k(dst3)


def _rowpass_kernel(hs, src3, dst3):
    @functools.partial(
        pl.kernel,
        out_type=jax.ShapeDtypeStruct((NC, NPAD, D), jnp.float32),
        mesh=_sc_mesh(),
        scratch_types=[
            pltpu.VMEM((CHUNK,), jnp.int32),
            pltpu.VMEM((CHUNK,), jnp.int32),
            pltpu.VMEM((CHUNK,), jnp.int32),
            pltpu.VMEM((CHUNK,), jnp.int32),
            pltpu.VMEM((CHUNK,), jnp.int32),
            pltpu.VMEM((CHUNK,), jnp.int32),
            pltpu.VMEM((CHUNK,), jnp.int32),
            pltpu.VMEM((CHUNK,), jnp.int32),
            pltpu.VMEM((CHUNK, D), jnp.float32),
            pltpu.VMEM((CHUNK, D), jnp.float32),
            pltpu.VMEM_SHARED((NPAD, D), jnp.float32),
            pltpu.SemaphoreType.DMA,
            pltpu.SemaphoreType.DMA,
            pltpu.SemaphoreType.DMA,
            pltpu.SemaphoreType.DMA,
            pltpu.SemaphoreType.DMA,
            pltpu.SemaphoreType.DMA,
        ],
    )
    def k(hs_ref, src_ref, dst_ref, out_ref, sv0, sv1, sv2, sv3,
          dv0, dv1, dv2, dv3, rows0, rows1, acc, is0, is1, is2, is3,
          gs0, gs1):
        src_v = (sv0, sv1, sv2, sv3)
        dst_v = (dv0, dv1, dv2, dv3)
        isems = (is0, is1, is2, is3)
        gsems = (gs0, gs1)
        c = lax.axis_index("c")
        s = lax.axis_index("s")
        wid = c * NS + s

        def fire_idx(g, sl):
            pltpu.async_copy(src_ref.at[wid, g], src_v[sl], isems[sl])
            pltpu.async_copy(dst_ref.at[wid, g], dst_v[sl], isems[sl])

        def wait_idx(g, sl):
            pltpu.make_async_copy(src_ref.at[wid, g], src_v[sl],
                                  isems[sl]).wait()
            pltpu.make_async_copy(dst_ref.at[wid, g], dst_v[sl],
                                  isems[sl]).wait()

        def pair(c0, q, fire_next_idx):
            g0, g1 = c0 + q, c0 + q + 1
            wait_idx(g0, q)
            wait_idx(g1, q + 1)
            cp0 = pltpu.async_copy(hs_ref.at[src_v[q]], rows0, gs0)
            cp1 = pltpu.async_copy(hs_ref.at[src_v[q + 1]], rows1, gs1)
            cp0.wait()
            cp1.wait()
            pltpu.sync_copy(rows0, acc.at[dst_v[q]], add=True)
            pltpu.sync_copy(rows1, acc.at[dst_v[q + 1]], add=True)
            if fire_next_idx:
                fire_idx(g0 + 4, q)
                fire_idx(g1 + 4, q + 1)

        @pl.loop(0, CHUNK)
        def _(i):
            @pl.loop(0, D // 16)
            def _(j):
                rows0[i, pl.ds(j * 16, 16)] = jnp.zeros((16,), jnp.float32)

        @pl.loop(0, SLAB // CHUNK)
        def _(j):
            pltpu.sync_copy(rows0,
                            acc.at[pl.ds(s * SLAB + j * CHUNK, CHUNK)])

        fire_idx(0, 0)
        fire_idx(1, 1)
        fire_idx(2, 2)
        fire_idx(3, 3)
        plsc.subcore_barrier()

        @pl.loop(0, NCHUNKS - 4, step=4)
        def _(c0):
            pair(c0, 0, True)
            pair(c0, 2, True)

        pair(NCHUNKS - 4, 0, False)
        pair(NCHUNKS - 4, 2, False)

        plsc.subcore_barrier()
        pltpu.sync_copy(acc.at[pl.ds(s * SLAB, SLAB)],
                        out_ref.at[c, pl.ds(s * SLAB, SLAB)])

    return k(hs, src3, dst3)


def _matmul_kernel(x, w):
    def body(x_ref, w_ref, o_ref):
        o_ref[...] = jnp.dot(x_ref[...], w_ref[...],
                             preferred_element_type=jnp.float32)

    return pl.pallas_call(
        body,
        grid=(NPAD // BLK,),
        in_specs=[
            pl.BlockSpec((BLK, D), lambda i: (i, 0)),
            pl.BlockSpec((D, D), lambda i: (0, 0)),
        ],
        out_specs=pl.BlockSpec((BLK, D), lambda i: (i, 0)),
        out_shape=jax.ShapeDtypeStruct((NPAD, D), jnp.float32),
    )(x, w)


def _scale_kernel(deg, h):
    def body(deg_ref, h_ref, dinv_ref, hs_ref):
        degsum = deg_ref[0, :, 0:1] + deg_ref[1, :, 0:1] + 1.0
        dinv = lax.rsqrt(degsum)
        dinv_rep = jnp.broadcast_to(dinv, (BLK, D))
        dinv_ref[...] = dinv_rep
        hs_ref[...] = dinv_rep * h_ref[...]

    return pl.pallas_call(
        body,
        grid=(NPAD // BLK,),
        in_specs=[
            pl.BlockSpec((NC, BLK, DEGW), lambda i: (0, i, 0)),
            pl.BlockSpec((BLK, D), lambda i: (i, 0)),
        ],
        out_specs=[
            pl.BlockSpec((BLK, D), lambda i: (i, 0)),
            pl.BlockSpec((BLK, D), lambda i: (i, 0)),
        ],
        out_shape=[
            jax.ShapeDtypeStruct((NPAD, D), jnp.float32),
            jax.ShapeDtypeStruct((NPAD, D), jnp.float32),
        ],
    )(deg, h)


def _layer_out_kernel(sp, hs, dinv, pre_b, w, post_b, scale_out):
    def body(sp_ref, hs_ref, dinv_ref, pb_ref, w_ref, qb_ref, o_ref):
        t = dinv_ref[...] * (sp_ref[0] + sp_ref[1] + hs_ref[...]) + pb_ref[...]
        r = jnp.maximum(t, 0.0)
        o = jnp.dot(r, w_ref[...], preferred_element_type=jnp.float32)
        if scale_out:
            o = dinv_ref[...] * o
        o_ref[...] = o + qb_ref[...]

    return pl.pallas_call(
        body,
        grid=(NPAD // BLK,),
        in_specs=[
            pl.BlockSpec((NC, BLK, D), lambda i: (0, i, 0)),
            pl.BlockSpec((BLK, D), lambda i: (i, 0)),
            pl.BlockSpec((BLK, D), lambda i: (i, 0)),
            pl.BlockSpec((1, D), lambda i: (0, 0)),
            pl.BlockSpec((D, D), lambda i: (0, 0)),
            pl.BlockSpec((1, D), lambda i: (0, 0)),
        ],
        out_specs=pl.BlockSpec((BLK, D), lambda i: (i, 0)),
        out_shape=jax.ShapeDtypeStruct((NPAD, D), jnp.float32),
    )(sp, hs, dinv, pre_b, w, post_b)


def kernel(x, edge_index, W1, b1, W2, b2, Wt, bt, We, be):
    src = edge_index[0].astype(jnp.int32)
    dst = edge_index[1].astype(jnp.int32)
    pad = (jnp.arange(EPAD - E, dtype=jnp.int32) % (NPAD - N)) + N
    src3 = jnp.concatenate([src, pad]).reshape(NW, NCHUNKS, CHUNK)
    dst3 = jnp.concatenate([dst, pad]).reshape(NW, NCHUNKS, CHUNK)

    xp = jnp.zeros((NPAD, D), jnp.float32).at[:N].set(x)
    b1r = b1.reshape(1, D)
    b2r = b2.reshape(1, D)
    zb = jnp.zeros((1, D), jnp.float32)
    Wh = jnp.zeros((D, D), jnp.float32).at[:, 0:1].set(Wt).at[:, 1:2].set(We)
    bh = jnp.zeros((1, D), jnp.float32).at[0, 0].set(bt[0]).at[0, 1].set(be[0])

    deg = _deg_kernel(dst3)
    h1 = _matmul_kernel(xp, W1)
    dinv, hs1 = _scale_kernel(deg, h1)
    s1 = _rowpass_kernel(hs1, src3, dst3)
    hs2 = _layer_out_kernel(s1, hs1, dinv, b1r, W2, zb, True)
    s2 = _rowpass_kernel(hs2, src3, dst3)
    out = _layer_out_kernel(s2, hs2, dinv, b2r, Wh, bh, False)

    return (out[:N, 0:1], out[:N, 1:2])


# speedup vs baseline: 25.1347x; 1.0618x over previous
import functools

import jax
import jax.numpy as jnp
from jax import lax
from jax.experimental import pallas as pl
from jax.experimental.pallas import tpu as pltpu
from jax.experimental.pallas import tpu_sc as plsc

N = 10000
NPAD = 10240
D = 128
E = 320000
NC = 2
NS = 16
NW = NC * NS
CHUNK = 128
NCHUNKS = 80
NBUF = 4
EPT = CHUNK * NCHUNKS
EPAD = NW * EPT
SLAB = NPAD // NS
DEGW = 16
BLK = 1024


def _sc_mesh():
    return plsc.VectorSubcoreMesh(core_axis_name="c", subcore_axis_name="s")


def _deg_kernel(dst3):
    @functools.partial(
        pl.kernel,
        out_type=jax.ShapeDtypeStruct((NC, NPAD, DEGW), jnp.float32),
        mesh=_sc_mesh(),
        scratch_types=[
            pltpu.VMEM((CHUNK,), jnp.int32),
            pltpu.VMEM((CHUNK,), jnp.int32),
            pltpu.VMEM((CHUNK, DEGW), jnp.float32),
            pltpu.VMEM((CHUNK, DEGW), jnp.float32),
            pltpu.VMEM_SHARED((NPAD, DEGW), jnp.float32),
            pltpu.SemaphoreType.DMA,
            pltpu.SemaphoreType.DMA,
        ],
    )
    def k(dst_ref, out_ref, iv0, iv1, ones_v, zeros_v, acc, is0, is1):
        idx_v = (iv0, iv1)
        isems = (is0, is1)
        c = lax.axis_index("c")
        s = lax.axis_index("s")
        wid = c * NS + s

        def fire_idx(g, sl):
            pltpu.async_copy(dst_ref.at[wid, g], idx_v[sl], isems[sl])

        def wait_idx(g, sl):
            pltpu.make_async_copy(dst_ref.at[wid, g], idx_v[sl],
                                  isems[sl]).wait()

        fire_idx(0, 0)
        fire_idx(1, 1)

        @pl.loop(0, CHUNK)
        def _(i):
            ones_v[i, :] = jnp.full((DEGW,), 1.0, jnp.float32)
            zeros_v[i, :] = jnp.zeros((DEGW,), jnp.float32)

        @pl.loop(0, SLAB // CHUNK)
        def _(j):
            pltpu.sync_copy(zeros_v, acc.at[pl.ds(s * SLAB + j * CHUNK, CHUNK)])

        plsc.subcore_barrier()

        @pl.loop(0, NCHUNKS, step=2)
        def _(c0):
            for b in range(2):
                g = c0 + b
                wait_idx(g, b)
                pltpu.sync_copy(ones_v, acc.at[idx_v[b]], add=True)

                @pl.when(g + 2 < NCHUNKS)
                def _():
                    fire_idx(g + 2, b)

        plsc.subcore_barrier()
        pltpu.sync_copy(acc.at[pl.ds(s * SLAB, SLAB)],
                        out_ref.at[c, pl.ds(s * SLAB, SLAB)])

    return k(dst3)


def _rowpass_kernel(hs, src3, dst3):
    @functools.partial(
        pl.kernel,
        out_type=jax.ShapeDtypeStruct((NC, NPAD, D), jnp.float32),
        mesh=_sc_mesh(),
        scratch_types=[
            pltpu.VMEM((CHUNK,), jnp.int32),
            pltpu.VMEM((CHUNK,), jnp.int32),
            pltpu.VMEM((CHUNK,), jnp.int32),
            pltpu.VMEM((CHUNK,), jnp.int32),
            pltpu.VMEM((CHUNK,), jnp.int32),
            pltpu.VMEM((CHUNK,), jnp.int32),
            pltpu.VMEM((CHUNK,), jnp.int32),
            pltpu.VMEM((CHUNK,), jnp.int32),
            pltpu.VMEM((CHUNK, D), jnp.float32),
            pltpu.VMEM((CHUNK, D), jnp.float32),
            pltpu.VMEM_SHARED((NPAD, D), jnp.float32),
            pltpu.SemaphoreType.DMA,
            pltpu.SemaphoreType.DMA,
            pltpu.SemaphoreType.DMA,
            pltpu.SemaphoreType.DMA,
            pltpu.SemaphoreType.DMA,
            pltpu.SemaphoreType.DMA,
        ],
    )
    def k(hs_ref, src_ref, dst_ref, out_ref, sv0, sv1, sv2, sv3,
          dv0, dv1, dv2, dv3, rows0, rows1, acc, is0, is1, is2, is3,
          gs0, gs1):
        src_v = (sv0, sv1, sv2, sv3)
        dst_v = (dv0, dv1, dv2, dv3)
        isems = (is0, is1, is2, is3)
        gsems = (gs0, gs1)
        c = lax.axis_index("c")
        s = lax.axis_index("s")
        wid = c * NS + s

        def fire_idx(g, sl):
            pltpu.async_copy(src_ref.at[wid, g], src_v[sl], isems[sl])
            pltpu.async_copy(dst_ref.at[wid, g], dst_v[sl], isems[sl])

        def wait_idx(g, sl):
            pltpu.make_async_copy(src_ref.at[wid, g], src_v[sl],
                                  isems[sl]).wait()
            pltpu.make_async_copy(dst_ref.at[wid, g], dst_v[sl],
                                  isems[sl]).wait()

        rows = (rows0, rows1)

        def quad(c0, fire_next_idx):
            wait_idx(c0, 0)
            cp = pltpu.async_copy(hs_ref.at[src_v[0]], rows0, gs0)
            for k in range(4):
                cp.wait()
                if k < 3:
                    wait_idx(c0 + k + 1, k + 1)
                    cp = pltpu.async_copy(hs_ref.at[src_v[k + 1]],
                                          rows[(k + 1) % 2],
                                          gsems[(k + 1) % 2])
                pltpu.sync_copy(rows[k % 2], acc.at[dst_v[k]], add=True)
                if fire_next_idx:
                    fire_idx(c0 + k + 4, k)

        @pl.loop(0, CHUNK)
        def _(i):
            @pl.loop(0, D // 16)
            def _(j):
                rows0[i, pl.ds(j * 16, 16)] = jnp.zeros((16,), jnp.float32)

        @pl.loop(0, SLAB // CHUNK)
        def _(j):
            pltpu.sync_copy(rows0,
                            acc.at[pl.ds(s * SLAB + j * CHUNK, CHUNK)])

        fire_idx(0, 0)
        fire_idx(1, 1)
        fire_idx(2, 2)
        fire_idx(3, 3)
        plsc.subcore_barrier()

        @pl.loop(0, NCHUNKS - 4, step=4)
        def _(c0):
            quad(c0, True)

        quad(NCHUNKS - 4, False)

        plsc.subcore_barrier()
        pltpu.sync_copy(acc.at[pl.ds(s * SLAB, SLAB)],
                        out_ref.at[c, pl.ds(s * SLAB, SLAB)])

    return k(hs, src3, dst3)


def _matmul_kernel(x, w):
    def body(x_ref, w_ref, o_ref):
        o_ref[...] = jnp.dot(x_ref[...], w_ref[...],
                             preferred_element_type=jnp.float32)

    return pl.pallas_call(
        body,
        grid=(NPAD // BLK,),
        in_specs=[
            pl.BlockSpec((BLK, D), lambda i: (i, 0)),
            pl.BlockSpec((D, D), lambda i: (0, 0)),
        ],
        out_specs=pl.BlockSpec((BLK, D), lambda i: (i, 0)),
        out_shape=jax.ShapeDtypeStruct((NPAD, D), jnp.float32),
    )(x, w)


def _scale_kernel(deg, h):
    def body(deg_ref, h_ref, dinv_ref, hs_ref):
        degsum = deg_ref[0, :, 0:1] + deg_ref[1, :, 0:1] + 1.0
        dinv = lax.rsqrt(degsum)
        dinv_rep = jnp.broadcast_to(dinv, (BLK, D))
        dinv_ref[...] = dinv_rep
        hs_ref[...] = dinv_rep * h_ref[...]

    return pl.pallas_call(
        body,
        grid=(NPAD // BLK,),
        in_specs=[
            pl.BlockSpec((NC, BLK, DEGW), lambda i: (0, i, 0)),
            pl.BlockSpec((BLK, D), lambda i: (i, 0)),
        ],
        out_specs=[
            pl.BlockSpec((BLK, D), lambda i: (i, 0)),
            pl.BlockSpec((BLK, D), lambda i: (i, 0)),
        ],
        out_shape=[
            jax.ShapeDtypeStruct((NPAD, D), jnp.float32),
            jax.ShapeDtypeStruct((NPAD, D), jnp.float32),
        ],
    )(deg, h)


def _layer_out_kernel(sp, hs, dinv, pre_b, w, post_b, scale_out):
    def body(sp_ref, hs_ref, dinv_ref, pb_ref, w_ref, qb_ref, o_ref):
        t = dinv_ref[...] * (sp_ref[0] + sp_ref[1] + hs_ref[...]) + pb_ref[...]
        r = jnp.maximum(t, 0.0)
        o = jnp.dot(r, w_ref[...], preferred_element_type=jnp.float32)
        if scale_out:
            o = dinv_ref[...] * o
        o_ref[...] = o + qb_ref[...]

    return pl.pallas_call(
        body,
        grid=(NPAD // BLK,),
        in_specs=[
            pl.BlockSpec((NC, BLK, D), lambda i: (0, i, 0)),
            pl.BlockSpec((BLK, D), lambda i: (i, 0)),
            pl.BlockSpec((BLK, D), lambda i: (i, 0)),
            pl.BlockSpec((1, D), lambda i: (0, 0)),
            pl.BlockSpec((D, D), lambda i: (0, 0)),
            pl.BlockSpec((1, D), lambda i: (0, 0)),
        ],
        out_specs=pl.BlockSpec((BLK, D), lambda i: (i, 0)),
        out_shape=jax.ShapeDtypeStruct((NPAD, D), jnp.float32),
    )(sp, hs, dinv, pre_b, w, post_b)


def kernel(x, edge_index, W1, b1, W2, b2, Wt, bt, We, be):
    src = edge_index[0].astype(jnp.int32)
    dst = edge_index[1].astype(jnp.int32)
    pad = (jnp.arange(EPAD - E, dtype=jnp.int32) % (NPAD - N)) + N
    src3 = jnp.concatenate([src, pad]).reshape(NW, NCHUNKS, CHUNK)
    dst3 = jnp.concatenate([dst, pad]).reshape(NW, NCHUNKS, CHUNK)

    xp = jnp.zeros((NPAD, D), jnp.float32).at[:N].set(x)
    b1r = b1.reshape(1, D)
    b2r = b2.reshape(1, D)
    zb = jnp.zeros((1, D), jnp.float32)
    Wh = jnp.zeros((D, D), jnp.float32).at[:, 0:1].set(Wt).at[:, 1:2].set(We)
    bh = jnp.zeros((1, D), jnp.float32).at[0, 0].set(bt[0]).at[0, 1].set(be[0])

    deg = _deg_kernel(dst3)
    h1 = _matmul_kernel(xp, W1)
    dinv, hs1 = _scale_kernel(deg, h1)
    s1 = _rowpass_kernel(hs1, src3, dst3)
    hs2 = _layer_out_kernel(s1, hs1, dinv, b1r, W2, zb, True)
    s2 = _rowpass_kernel(hs2, src3, dst3)
    out = _layer_out_kernel(s2, hs2, dinv, b2r, Wh, bh, False)

    return (out[:N, 0:1], out[:N, 1:2])


# speedup vs baseline: 28.1813x; 1.1212x over previous
import functools

import jax
import jax.numpy as jnp
from jax import lax
from jax.experimental import pallas as pl
from jax.experimental.pallas import tpu as pltpu
from jax.experimental.pallas import tpu_sc as plsc

N = 10000
NPAD = 10240
D = 128
E = 320000
NC = 2
NS = 16
NW = NC * NS
CHUNK = 128
NCHUNKS = 80
NBUF = 4
EPT = CHUNK * NCHUNKS
EPAD = NW * EPT
SLAB = NPAD // NS
DEGW = 16
BLK = 1024


def _sc_mesh():
    return plsc.VectorSubcoreMesh(core_axis_name="c", subcore_axis_name="s")


def _deg_kernel(dst3):
    @functools.partial(
        pl.kernel,
        out_type=jax.ShapeDtypeStruct((NC, NPAD, DEGW), jnp.float32),
        mesh=_sc_mesh(),
        scratch_types=[
            pltpu.VMEM((CHUNK,), jnp.int32),
            pltpu.VMEM((CHUNK,), jnp.int32),
            pltpu.VMEM((CHUNK, DEGW), jnp.float32),
            pltpu.VMEM((CHUNK, DEGW), jnp.float32),
            pltpu.VMEM_SHARED((NPAD, DEGW), jnp.float32),
            pltpu.SemaphoreType.DMA,
            pltpu.SemaphoreType.DMA,
        ],
    )
    def k(dst_ref, out_ref, iv0, iv1, ones_v, zeros_v, acc, is0, is1):
        idx_v = (iv0, iv1)
        isems = (is0, is1)
        c = lax.axis_index("c")
        s = lax.axis_index("s")
        wid = c * NS + s

        def fire_idx(g, sl):
            pltpu.async_copy(dst_ref.at[wid, g], idx_v[sl], isems[sl])

        def wait_idx(g, sl):
            pltpu.make_async_copy(dst_ref.at[wid, g], idx_v[sl],
                                  isems[sl]).wait()

        fire_idx(0, 0)
        fire_idx(1, 1)

        @pl.loop(0, CHUNK)
        def _(i):
            ones_v[i, :] = jnp.full((DEGW,), 1.0, jnp.float32)
            zeros_v[i, :] = jnp.zeros((DEGW,), jnp.float32)

        @pl.loop(0, SLAB // CHUNK)
        def _(j):
            pltpu.sync_copy(zeros_v, acc.at[pl.ds(s * SLAB + j * CHUNK, CHUNK)])

        plsc.subcore_barrier()

        @pl.loop(0, NCHUNKS, step=2)
        def _(c0):
            for b in range(2):
                g = c0 + b
                wait_idx(g, b)
                pltpu.sync_copy(ones_v, acc.at[idx_v[b]], add=True)

                @pl.when(g + 2 < NCHUNKS)
                def _():
                    fire_idx(g + 2, b)

        plsc.subcore_barrier()
        pltpu.sync_copy(acc.at[pl.ds(s * SLAB, SLAB)],
                        out_ref.at[c, pl.ds(s * SLAB, SLAB)])

    return k(dst3)


def _rowpass_kernel(hs, src3, dst3):
    @functools.partial(
        pl.kernel,
        out_type=jax.ShapeDtypeStruct((NC, NPAD, D), jnp.float32),
        mesh=_sc_mesh(),
    scratch_types=(
            [pltpu.VMEM((CHUNK,), jnp.int32)] * 16
            + [pltpu.VMEM((CHUNK, D), jnp.float32)] * 2
            + [pltpu.VMEM_SHARED((NPAD, D), jnp.float32)]
            + [pltpu.SemaphoreType.DMA] * 10
        ),
    )
    def k(hs_ref, src_ref, dst_ref, out_ref, *sc):
        src_v = sc[0:8]
        dst_v = sc[8:16]
        rows0, rows1 = sc[16], sc[17]
        acc = sc[18]
        isems = sc[19:27]
        gsems = (sc[27], sc[28])
        gs0 = gsems[0]
        c = lax.axis_index("c")
        s = lax.axis_index("s")
        wid = c * NS + s

        def fire_idx(g, sl):
            pltpu.async_copy(src_ref.at[wid, g], src_v[sl], isems[sl])
            pltpu.async_copy(dst_ref.at[wid, g], dst_v[sl], isems[sl])

        def wait_idx(g, sl):
            pltpu.make_async_copy(src_ref.at[wid, g], src_v[sl],
                                  isems[sl]).wait()
            pltpu.make_async_copy(dst_ref.at[wid, g], dst_v[sl],
                                  isems[sl]).wait()

        rows = (rows0, rows1)
        GRP = 8

        def grp(c0, fire_next_idx):
            cp = [None] * GRP
            for k in range(2):
                wait_idx(c0 + k, k)
                cp[k] = pltpu.async_copy(hs_ref.at[src_v[k]], rows[k % 2],
                                         gsems[k % 2])
            for k in range(GRP):
                cp[k].wait()
                pltpu.sync_copy(rows[k % 2], acc.at[dst_v[k]], add=True)
                if k < GRP - 2:
                    wait_idx(c0 + k + 2, k + 2)
                    cp[k + 2] = pltpu.async_copy(hs_ref.at[src_v[k + 2]],
                                                 rows[k % 2], gsems[k % 2])
                if fire_next_idx:
                    fire_idx(c0 + k + GRP, k)

        @pl.loop(0, CHUNK)
        def _(i):
            @pl.loop(0, D // 16)
            def _(j):
                rows0[i, pl.ds(j * 16, 16)] = jnp.zeros((16,), jnp.float32)

        @pl.loop(0, SLAB // CHUNK)
        def _(j):
            pltpu.sync_copy(rows0,
                            acc.at[pl.ds(s * SLAB + j * CHUNK, CHUNK)])

        for sl in range(GRP):
            fire_idx(sl, sl)
        plsc.subcore_barrier()

        @pl.loop(0, NCHUNKS - GRP, step=GRP)
        def _(c0):
            grp(c0, True)

        grp(NCHUNKS - GRP, False)

        plsc.subcore_barrier()
        pltpu.sync_copy(acc.at[pl.ds(s * SLAB, SLAB)],
                        out_ref.at[c, pl.ds(s * SLAB, SLAB)])

    return k(hs, src3, dst3)


def _matmul_kernel(x, w):
    def body(x_ref, w_ref, o_ref):
        o_ref[...] = jnp.dot(x_ref[...], w_ref[...],
                             preferred_element_type=jnp.float32)

    return pl.pallas_call(
        body,
        grid=(NPAD // BLK,),
        in_specs=[
            pl.BlockSpec((BLK, D), lambda i: (i, 0)),
            pl.BlockSpec((D, D), lambda i: (0, 0)),
        ],
        out_specs=pl.BlockSpec((BLK, D), lambda i: (i, 0)),
        out_shape=jax.ShapeDtypeStruct((NPAD, D), jnp.float32),
    )(x, w)


def _scale_kernel(deg, h):
    def body(deg_ref, h_ref, dinv_ref, hs_ref):
        degsum = deg_ref[0, :, 0:1] + deg_ref[1, :, 0:1] + 1.0
        dinv = lax.rsqrt(degsum)
        dinv_rep = jnp.broadcast_to(dinv, (BLK, D))
        dinv_ref[...] = dinv_rep
        hs_ref[...] = dinv_rep * h_ref[...]

    return pl.pallas_call(
        body,
        grid=(NPAD // BLK,),
        in_specs=[
            pl.BlockSpec((NC, BLK, DEGW), lambda i: (0, i, 0)),
            pl.BlockSpec((BLK, D), lambda i: (i, 0)),
        ],
        out_specs=[
            pl.BlockSpec((BLK, D), lambda i: (i, 0)),
            pl.BlockSpec((BLK, D), lambda i: (i, 0)),
        ],
        out_shape=[
            jax.ShapeDtypeStruct((NPAD, D), jnp.float32),
            jax.ShapeDtypeStruct((NPAD, D), jnp.float32),
        ],
    )(deg, h)


def _layer_out_kernel(sp, hs, dinv, pre_b, w, post_b, scale_out):
    def body(sp_ref, hs_ref, dinv_ref, pb_ref, w_ref, qb_ref, o_ref):
        t = dinv_ref[...] * (sp_ref[0] + sp_ref[1] + hs_ref[...]) + pb_ref[...]
        r = jnp.maximum(t, 0.0)
        o = jnp.dot(r, w_ref[...], preferred_element_type=jnp.float32)
        if scale_out:
            o = dinv_ref[...] * o
        o_ref[...] = o + qb_ref[...]

    return pl.pallas_call(
        body,
        grid=(NPAD // BLK,),
        in_specs=[
            pl.BlockSpec((NC, BLK, D), lambda i: (0, i, 0)),
            pl.BlockSpec((BLK, D), lambda i: (i, 0)),
            pl.BlockSpec((BLK, D), lambda i: (i, 0)),
            pl.BlockSpec((1, D), lambda i: (0, 0)),
            pl.BlockSpec((D, D), lambda i: (0, 0)),
            pl.BlockSpec((1, D), lambda i: (0, 0)),
        ],
        out_specs=pl.BlockSpec((BLK, D), lambda i: (i, 0)),
        out_shape=jax.ShapeDtypeStruct((NPAD, D), jnp.float32),
    )(sp, hs, dinv, pre_b, w, post_b)


def kernel(x, edge_index, W1, b1, W2, b2, Wt, bt, We, be):
    src = edge_index[0].astype(jnp.int32)
    dst = edge_index[1].astype(jnp.int32)
    pad = (jnp.arange(EPAD - E, dtype=jnp.int32) % (NPAD - N)) + N
    src3 = jnp.concatenate([src, pad]).reshape(NW, NCHUNKS, CHUNK)
    dst3 = jnp.concatenate([dst, pad]).reshape(NW, NCHUNKS, CHUNK)

    xp = jnp.zeros((NPAD, D), jnp.float32).at[:N].set(x)
    b1r = b1.reshape(1, D)
    b2r = b2.reshape(1, D)
    zb = jnp.zeros((1, D), jnp.float32)
    Wh = jnp.zeros((D, D), jnp.float32).at[:, 0:1].set(Wt).at[:, 1:2].set(We)
    bh = jnp.zeros((1, D), jnp.float32).at[0, 0].set(bt[0]).at[0, 1].set(be[0])

    deg = _deg_kernel(dst3)
    h1 = _matmul_kernel(xp, W1)
    dinv, hs1 = _scale_kernel(deg, h1)
    s1 = _rowpass_kernel(hs1, src3, dst3)
    hs2 = _layer_out_kernel(s1, hs1, dinv, b1r, W2, zb, True)
    s2 = _rowpass_kernel(hs2, src3, dst3)
    out = _layer_out_kernel(s2, hs2, dinv, b2r, Wh, bh, False)

    return (out[:N, 0:1], out[:N, 1:2])


# speedup vs baseline: 29.5655x; 1.0491x over previous
import functools

import jax
import jax.numpy as jnp
from jax import lax
from jax.experimental import pallas as pl
from jax.experimental.pallas import tpu as pltpu
from jax.experimental.pallas import tpu_sc as plsc

N = 10000
NPAD = 10240
D = 128
E = 320000
NC = 2
NS = 16
NW = NC * NS
CHUNK = 128
NCHUNKS = 80
NBUF = 4
EPT = CHUNK * NCHUNKS
EPAD = NW * EPT
SLAB = NPAD // NS
DEGW = 16
BLK = 1024


def _sc_mesh():
    return plsc.VectorSubcoreMesh(core_axis_name="c", subcore_axis_name="s")


def _deg_kernel(dst3):
    @functools.partial(
        pl.kernel,
        out_type=jax.ShapeDtypeStruct((NC, NPAD, DEGW), jnp.float32),
        mesh=_sc_mesh(),
        scratch_types=(
            [pltpu.VMEM((CHUNK,), jnp.int32)] * 8
            + [pltpu.VMEM((CHUNK, DEGW), jnp.float32)] * 2
            + [pltpu.VMEM_SHARED((NPAD, DEGW), jnp.float32)]
            + [pltpu.SemaphoreType.DMA] * 10
        ),
    )
    def k(dst_ref, out_ref, *sc):
        idx_v = sc[0:8]
        ones_v, zeros_v = sc[8], sc[9]
        acc = sc[10]
        isems = sc[11:19]
        ssems = (sc[19], sc[20])
        c = lax.axis_index("c")
        s = lax.axis_index("s")
        wid = c * NS + s
        GRP = 8

        def fire_idx(g, sl):
            pltpu.async_copy(dst_ref.at[wid, g], idx_v[sl], isems[sl])

        def wait_idx(g, sl):
            pltpu.make_async_copy(dst_ref.at[wid, g], idx_v[sl],
                                  isems[sl]).wait()

        def grp(c0, fire_next_idx):
            sp = [None] * GRP
            for k in range(GRP):
                wait_idx(c0 + k, k)
                sp[k] = pltpu.async_copy(ones_v, acc.at[idx_v[k]],
                                         ssems[k % 2], add=True)
                if k >= 1:
                    sp[k - 1].wait()
                    if fire_next_idx:
                        fire_idx(c0 + (k - 1) + GRP, k - 1)
            sp[GRP - 1].wait()
            if fire_next_idx:
                fire_idx(c0 + (GRP - 1) + GRP, GRP - 1)

        for sl in range(GRP):
            fire_idx(sl, sl)

        @pl.loop(0, CHUNK)
        def _(i):
            ones_v[i, :] = jnp.full((DEGW,), 1.0, jnp.float32)
            zeros_v[i, :] = jnp.zeros((DEGW,), jnp.float32)

        @pl.loop(0, SLAB // CHUNK)
        def _(j):
            pltpu.sync_copy(zeros_v, acc.at[pl.ds(s * SLAB + j * CHUNK, CHUNK)])

        plsc.subcore_barrier()

        @pl.loop(0, NCHUNKS - GRP, step=GRP)
        def _(c0):
            grp(c0, True)

        grp(NCHUNKS - GRP, False)

        plsc.subcore_barrier()
        pltpu.sync_copy(acc.at[pl.ds(s * SLAB, SLAB)],
                        out_ref.at[c, pl.ds(s * SLAB, SLAB)])

    return k(dst3)


def _rowpass_kernel(hs, src3, dst3):
    @functools.partial(
        pl.kernel,
        out_type=jax.ShapeDtypeStruct((NC, NPAD, D), jnp.float32),
        mesh=_sc_mesh(),
    scratch_types=(
            [pltpu.VMEM((CHUNK,), jnp.int32)] * 16
            + [pltpu.VMEM((CHUNK, D), jnp.float32)] * 2
            + [pltpu.VMEM_SHARED((NPAD, D), jnp.float32)]
            + [pltpu.SemaphoreType.DMA] * 10
        ),
    )
    def k(hs_ref, src_ref, dst_ref, out_ref, *sc):
        src_v = sc[0:8]
        dst_v = sc[8:16]
        rows0, rows1 = sc[16], sc[17]
        acc = sc[18]
        isems = sc[19:27]
        gsems = (sc[27], sc[28])
        gs0 = gsems[0]
        c = lax.axis_index("c")
        s = lax.axis_index("s")
        wid = c * NS + s

        def fire_idx(g, sl):
            pltpu.async_copy(src_ref.at[wid, g], src_v[sl], isems[sl])
            pltpu.async_copy(dst_ref.at[wid, g], dst_v[sl], isems[sl])

        def wait_idx(g, sl):
            pltpu.make_async_copy(src_ref.at[wid, g], src_v[sl],
                                  isems[sl]).wait()
            pltpu.make_async_copy(dst_ref.at[wid, g], dst_v[sl],
                                  isems[sl]).wait()

        rows = (rows0, rows1)
        GRP = 8

        def grp(c0, fire_next_idx):
            cp = [None] * GRP
            for k in range(2):
                wait_idx(c0 + k, k)
                cp[k] = pltpu.async_copy(hs_ref.at[src_v[k]], rows[k % 2],
                                         gsems[k % 2])
            for k in range(GRP):
                cp[k].wait()
                pltpu.sync_copy(rows[k % 2], acc.at[dst_v[k]], add=True)
                if k < GRP - 2:
                    wait_idx(c0 + k + 2, k + 2)
                    cp[k + 2] = pltpu.async_copy(hs_ref.at[src_v[k + 2]],
                                                 rows[k % 2], gsems[k % 2])
                if fire_next_idx:
                    fire_idx(c0 + k + GRP, k)

        @pl.loop(0, CHUNK)
        def _(i):
            @pl.loop(0, D // 16)
            def _(j):
                rows0[i, pl.ds(j * 16, 16)] = jnp.zeros((16,), jnp.float32)

        @pl.loop(0, SLAB // CHUNK)
        def _(j):
            pltpu.sync_copy(rows0,
                            acc.at[pl.ds(s * SLAB + j * CHUNK, CHUNK)])

        for sl in range(GRP):
            fire_idx(sl, sl)
        plsc.subcore_barrier()

        @pl.loop(0, NCHUNKS - GRP, step=GRP)
        def _(c0):
            grp(c0, True)

        grp(NCHUNKS - GRP, False)

        plsc.subcore_barrier()
        pltpu.sync_copy(acc.at[pl.ds(s * SLAB, SLAB)],
                        out_ref.at[c, pl.ds(s * SLAB, SLAB)])

    return k(hs, src3, dst3)


def _matmul_kernel(x, w):
    def body(x_ref, w_ref, o_ref):
        o_ref[...] = jnp.dot(x_ref[...], w_ref[...],
                             preferred_element_type=jnp.float32)

    return pl.pallas_call(
        body,
        grid=(NPAD // BLK,),
        in_specs=[
            pl.BlockSpec((BLK, D), lambda i: (i, 0)),
            pl.BlockSpec((D, D), lambda i: (0, 0)),
        ],
        out_specs=pl.BlockSpec((BLK, D), lambda i: (i, 0)),
        out_shape=jax.ShapeDtypeStruct((NPAD, D), jnp.float32),
    )(x, w)


def _scale_kernel(deg, h):
    def body(deg_ref, h_ref, dinv_ref, hs_ref):
        degsum = deg_ref[0, :, 0:1] + deg_ref[1, :, 0:1] + 1.0
        dinv = lax.rsqrt(degsum)
        dinv_rep = jnp.broadcast_to(dinv, (BLK, D))
        dinv_ref[...] = dinv_rep
        hs_ref[...] = dinv_rep * h_ref[...]

    return pl.pallas_call(
        body,
        grid=(NPAD // BLK,),
        in_specs=[
            pl.BlockSpec((NC, BLK, DEGW), lambda i: (0, i, 0)),
            pl.BlockSpec((BLK, D), lambda i: (i, 0)),
        ],
        out_specs=[
            pl.BlockSpec((BLK, D), lambda i: (i, 0)),
            pl.BlockSpec((BLK, D), lambda i: (i, 0)),
        ],
        out_shape=[
            jax.ShapeDtypeStruct((NPAD, D), jnp.float32),
            jax.ShapeDtypeStruct((NPAD, D), jnp.float32),
        ],
    )(deg, h)


def _layer_out_kernel(sp, hs, dinv, pre_b, w, post_b, scale_out):
    def body(sp_ref, hs_ref, dinv_ref, pb_ref, w_ref, qb_ref, o_ref):
        t = dinv_ref[...] * (sp_ref[0] + sp_ref[1] + hs_ref[...]) + pb_ref[...]
        r = jnp.maximum(t, 0.0)
        o = jnp.dot(r, w_ref[...], preferred_element_type=jnp.float32)
        if scale_out:
            o = dinv_ref[...] * o
        o_ref[...] = o + qb_ref[...]

    return pl.pallas_call(
        body,
        grid=(NPAD // BLK,),
        in_specs=[
            pl.BlockSpec((NC, BLK, D), lambda i: (0, i, 0)),
            pl.BlockSpec((BLK, D), lambda i: (i, 0)),
            pl.BlockSpec((BLK, D), lambda i: (i, 0)),
            pl.BlockSpec((1, D), lambda i: (0, 0)),
            pl.BlockSpec((D, D), lambda i: (0, 0)),
            pl.BlockSpec((1, D), lambda i: (0, 0)),
        ],
        out_specs=pl.BlockSpec((BLK, D), lambda i: (i, 0)),
        out_shape=jax.ShapeDtypeStruct((NPAD, D), jnp.float32),
    )(sp, hs, dinv, pre_b, w, post_b)


def kernel(x, edge_index, W1, b1, W2, b2, Wt, bt, We, be):
    src = edge_index[0].astype(jnp.int32)
    dst = edge_index[1].astype(jnp.int32)
    pad = (jnp.arange(EPAD - E, dtype=jnp.int32) % (NPAD - N)) + N
    src3 = jnp.concatenate([src, pad]).reshape(NW, NCHUNKS, CHUNK)
    dst3 = jnp.concatenate([dst, pad]).reshape(NW, NCHUNKS, CHUNK)

    xp = jnp.zeros((NPAD, D), jnp.float32).at[:N].set(x)
    b1r = b1.reshape(1, D)
    b2r = b2.reshape(1, D)
    zb = jnp.zeros((1, D), jnp.float32)
    Wh = jnp.zeros((D, D), jnp.float32).at[:, 0:1].set(Wt).at[:, 1:2].set(We)
    bh = jnp.zeros((1, D), jnp.float32).at[0, 0].set(bt[0]).at[0, 1].set(be[0])

    deg = _deg_kernel(dst3)
    h1 = _matmul_kernel(xp, W1)
    dinv, hs1 = _scale_kernel(deg, h1)
    s1 = _rowpass_kernel(hs1, src3, dst3)
    hs2 = _layer_out_kernel(s1, hs1, dinv, b1r, W2, zb, True)
    s2 = _rowpass_kernel(hs2, src3, dst3)
    out = _layer_out_kernel(s2, hs2, dinv, b2r, Wh, bh, False)

    return (out[:N, 0:1], out[:N, 1:2])
